# baseline (speedup 1.0000x reference)
import functools
import math

import jax
import jax.numpy as jnp
from jax import lax
from jax.experimental import pallas as pl
from jax.experimental.pallas import tpu as pltpu
from jax.experimental.pallas import tpu_sc as plsc

_B = 16
_T = 1500
_TP = 1536
_W = 512
_NH = 8
_HW = 64
_NMLP = 2048
_CODES = 512
_CBD = 32
_QK_SCALE = 2.0 * 8 / math.sqrt(_HW)
_ROWS = _B * _T // 2
_ROWS_PAD = 12032
_NW = 32
_BPW = _ROWS_PAD // _NW

_HI = lax.Precision.HIGHEST


def _gelu(x):
    return 0.5 * x * (1.0 + lax.erf(x * (1.0 / math.sqrt(2.0))))


def _ln_rows(x, g, b):
    m = jnp.mean(x, axis=-1, keepdims=True)
    xc = x - m
    v = jnp.mean(xc * xc, axis=-1, keepdims=True)
    return xc / jnp.sqrt(v + 1e-5) * g + b


def _k1_body(e0_ref, e1_ref, w1_ref, b1_ref, w2_ref, b2_ref, g_ref, bb_ref,
             pw_ref, pb_ref, cbt_ref, z_ref, idx_ref):
    x = (e0_ref[...] + e1_ref[...]) * 0.5
    h = _ln_rows(x, g_ref[...], bb_ref[...])
    a = jnp.dot(h, w1_ref[...]) + b1_ref[...]
    a = _gelu(a)
    x = x + jnp.dot(a, w2_ref[...]) + b2_ref[...]
    z = jnp.dot(x, pw_ref[...]) + pb_ref[...]
    cbt = cbt_ref[...]
    zc = jnp.dot(z, cbt)
    cb2 = jnp.sum(cbt * cbt, axis=0, keepdims=True)
    d = cb2 - 2.0 * zc
    md = jnp.min(d, axis=1, keepdims=True)
    ii = lax.broadcasted_iota(jnp.int32, d.shape, 1)
    idx = jnp.min(jnp.where(d == md, ii, _CODES), axis=1)
    z_ref[...] = z
    idx_ref[...] = idx[:, None]


def _k1(e0, e1, mlp_w1, mlp_b1, mlp_w2, mlp_b2, mlp_ln_g, mlp_ln_b,
        proj_in_w, proj_in_b, cbt):
    rt = 600
    ntile = _ROWS // rt
    full = lambda shape: pl.BlockSpec(shape, lambda i: (0,) * len(shape))
    return pl.pallas_call(
        _k1_body,
        grid=(ntile,),
        in_specs=[
            pl.BlockSpec((rt, _W), lambda i: (i, 0)),
            pl.BlockSpec((rt, _W), lambda i: (i, 0)),
            full((_W, _NMLP)),
            full((1, _NMLP)),
            full((_NMLP, _W)),
            full((1, _W)),
            full((1, _W)),
            full((1, _W)),
            full((_W, _CBD)),
            full((1, _CBD)),
            full((_CBD, _CODES)),
        ],
        out_specs=[
            pl.BlockSpec((rt, _CBD), lambda i: (i, 0)),
            pl.BlockSpec((rt, 1), lambda i: (i, 0)),
        ],
        out_shape=[
            jax.ShapeDtypeStruct((_ROWS, _CBD), jnp.float32),
            jax.ShapeDtypeStruct((_ROWS, 1), jnp.int32),
        ],
    )(e0, e1, mlp_w1, mlp_b1[None, :], mlp_w2, mlp_b2[None, :],
      mlp_ln_g[None, :], mlp_ln_b[None, :], proj_in_w, proj_in_b[None, :],
      cbt)


_GW = 128


def _sc_gather_body(table_hbm, idx_hbm, out_hbm, idx_v, rows_v, sem):
    wid = lax.axis_index("s") * 2 + lax.axis_index("c")
    base = wid * _BPW
    pltpu.sync_copy(idx_hbm.at[pl.ds(base, _BPW)], idx_v)
    pltpu.async_copy(table_hbm.at[idx_v], rows_v, sem).wait()
    pltpu.sync_copy(rows_v, out_hbm.at[pl.ds(base, _BPW)])


def _gather_codes(codebook, idxp):
    table = jnp.pad(codebook, ((0, 0), (0, _GW - _CBD)))
    mesh = plsc.VectorSubcoreMesh(core_axis_name="c", subcore_axis_name="s")
    fn = pl.kernel(
        _sc_gather_body,
        out_type=jax.ShapeDtypeStruct((_ROWS_PAD, _GW), jnp.float32),
        mesh=mesh,
        scratch_types=[
            pltpu.VMEM((_BPW,), jnp.int32),
            pltpu.VMEM((_BPW, _GW), jnp.float32),
            pltpu.SemaphoreType.DMA,
        ],
    )
    return fn(table, idxp)[:, :_CBD]


def _k3_body(qf_ref, z_ref, wpo_ref, pbo_ref, quant_ref, closs_ref):
    i = pl.program_id(0)
    qf = qf_ref[...]
    quant_ref[...] = jnp.dot(qf, wpo_ref[...], precision=_HI) + pbo_ref[...]
    rt = qf.shape[0]
    rows = i * rt + lax.broadcasted_iota(jnp.int32, (rt, 1), 0)
    w = (rows < _ROWS).astype(jnp.float32)
    dz = z_ref[...] - qf
    c = jnp.sum(w * dz * dz)
    closs_ref[...] = jnp.broadcast_to(c, (1, 1, 128))


def _k3(qfp, zp, proj_out_w, proj_out_b):
    rt = 752
    ntile = _ROWS_PAD // rt
    return pl.pallas_call(
        _k3_body,
        grid=(ntile,),
        in_specs=[
            pl.BlockSpec((rt, _CBD), lambda i: (i, 0)),
            pl.BlockSpec((rt, _CBD), lambda i: (i, 0)),
            pl.BlockSpec((_CBD, _W), lambda i: (0, 0)),
            pl.BlockSpec((1, _W), lambda i: (0, 0)),
        ],
        out_specs=[
            pl.BlockSpec((rt, _W), lambda i: (i, 0)),
            pl.BlockSpec((1, 1, 128), lambda i: (i, 0, 0)),
        ],
        out_shape=[
            jax.ShapeDtypeStruct((_ROWS_PAD, _W), jnp.float32),
            jax.ShapeDtypeStruct((ntile, 1, 128), jnp.float32),
        ],
    )(qfp, zp, proj_out_w, proj_out_b[None, :])


def _k4_body(up_ref, mk_ref, pos_ref, mv_ref, g_ref, b_ref,
             wq_ref, bq_ref, wk_ref, wv_ref, bv_ref,
             xa_ref, q_ref, k_ref, v_ref):
    up = up_ref[0]
    mk = mk_ref[0]
    xa = jnp.where(mk > 0, up, mv_ref[...]) + pos_ref[...]
    xa_ref[0] = xa
    h = _ln_rows(xa, g_ref[...], b_ref[...])
    q_ref[0] = (jnp.dot(h, wq_ref[...], precision=_HI) + bq_ref[...]) * _QK_SCALE
    k_ref[0] = jnp.dot(h, wk_ref[...], precision=_HI)
    v_ref[0] = jnp.dot(h, wv_ref[...], precision=_HI) + bv_ref[...]


def _k4(upp, maskf, posp, mv, attn_ln_g, attn_ln_b, wq, bq, wk, wv, bv):
    tt = 512
    nt = _TP // tt
    full2 = lambda shape: pl.BlockSpec(shape, lambda b, j: (0,) * len(shape))
    o_spec = pl.BlockSpec((1, tt, _W), lambda b, j: (b, j, 0))
    o_shape = jax.ShapeDtypeStruct((_B, _TP, _W), jnp.float32)
    return pl.pallas_call(
        _k4_body,
        grid=(_B, nt),
        in_specs=[
            pl.BlockSpec((1, tt, _W), lambda b, j: (b, j, 0)),
            pl.BlockSpec((1, tt, 1), lambda b, j: (b, j, 0)),
            pl.BlockSpec((tt, _W), lambda b, j: (j, 0)),
            full2((1, _W)),
            full2((1, _W)),
            full2((1, _W)),
            full2((_W, _W)),
            full2((1, _W)),
            full2((_W, _W)),
            full2((_W, _W)),
            full2((1, _W)),
        ],
        out_specs=[o_spec, o_spec, o_spec, o_spec],
        out_shape=[o_shape, o_shape, o_shape, o_shape],
    )(upp, maskf, posp, mv, attn_ln_g[None, :], attn_ln_b[None, :],
      wq, bq[None, :], wk, wv, bv[None, :])


def _k5_body(q_ref, k_ref, v_ref, cos_ref, sin_ref, o_ref):
    cos = cos_ref[...]
    sin = sin_ref[...]

    def rope(x):
        x1 = x[:, :_HW // 2]
        x2 = x[:, _HW // 2:]
        return jnp.concatenate([x1 * cos - x2 * sin, x1 * sin + x2 * cos],
                               axis=1)

    qr = rope(q_ref[0, 0])
    kr = rope(k_ref[0, 0])
    s = jnp.dot(qr, kr.T, precision=_HI)
    col = lax.broadcasted_iota(jnp.int32, s.shape, 1)
    s = jnp.where(col < _T, s, -1e30)
    m = jnp.max(s, axis=1, keepdims=True)
    p = jnp.exp(s - m)
    ps = jnp.sum(p, axis=1, keepdims=True)
    o = jnp.dot(p, v_ref[0, 0], precision=_HI)
    o_ref[0, 0] = o / ps


def _k5(q4, k4, v4, cos, sin):
    spec = pl.BlockSpec((1, 1, _TP, _HW), lambda b, h: (b, h, 0, 0))
    return pl.pallas_call(
        _k5_body,
        grid=(_B, _NH),
        in_specs=[
            spec, spec, spec,
            pl.BlockSpec((_TP, _HW // 2), lambda b, h: (0, 0)),
            pl.BlockSpec((_TP, _HW // 2), lambda b, h: (0, 0)),
        ],
        out_specs=spec,
        out_shape=jax.ShapeDtypeStruct((_B, _NH, _TP, _HW), jnp.float32),
    )(q4, k4, v4, cos, sin)


def _k6_body(xa_ref, o_ref, wo_ref, bo_ref, g1_ref, b1_ref,
             fw1_ref, fb1_ref, fw2_ref, fb2_ref, gp_ref, bp_ref, out_ref):
    x = xa_ref[0] + jnp.dot(o_ref[0], wo_ref[...], precision=_HI) + bo_ref[...]
    h = _ln_rows(x, g1_ref[...], b1_ref[...])
    a = _gelu(jnp.dot(h, fw1_ref[...], precision=_HI) + fb1_ref[...])
    x = x + jnp.dot(a, fw2_ref[...], precision=_HI) + fb2_ref[...]
    out_ref[0] = _ln_rows(x, gp_ref[...], bp_ref[...])


def _k6(xa, o, wo, bo, ffn_ln_g, ffn_ln_b, ffn_w1, ffn_b1, ffn_w2, ffn_b2,
        ln_post_g, ln_post_b):
    tt = 512
    nt = _TP // tt
    full2 = lambda shape: pl.BlockSpec(shape, lambda b, j: (0,) * len(shape))
    io_spec = pl.BlockSpec((1, tt, _W), lambda b, j: (b, j, 0))
    return pl.pallas_call(
        _k6_body,
        grid=(_B, nt),
        in_specs=[
            io_spec, io_spec,
            full2((_W, _W)),
            full2((1, _W)),
            full2((1, _W)),
            full2((1, _W)),
            full2((_W, _NMLP)),
            full2((1, _NMLP)),
            full2((_NMLP, _W)),
            full2((1, _W)),
            full2((1, _W)),
            full2((1, _W)),
        ],
        out_specs=io_spec,
        out_shape=jax.ShapeDtypeStruct((_B, _TP, _W), jnp.float32),
    )(xa, o, wo, bo[None, :], ffn_ln_g[None, :], ffn_ln_b[None, :],
      ffn_w1, ffn_b1[None, :], ffn_w2, ffn_b2[None, :],
      ln_post_g[None, :], ln_post_b[None, :])


def kernel(embs, mask, mlp_w1, mlp_b1, mlp_w2, mlp_b2, mlp_ln_g, mlp_ln_b,
           proj_in_w, proj_in_b, codebook, proj_out_w, proj_out_b, pos_emb,
           attn_ln_g, attn_ln_b, wq, bq, wk, wv, bv, wo, bo,
           ffn_ln_g, ffn_ln_b, ffn_w1, ffn_b1, ffn_w2, ffn_b2,
           ln_post_g, ln_post_b):
    e0 = embs[:, 0::2, :].reshape(_ROWS, _W)
    e1 = embs[:, 1::2, :].reshape(_ROWS, _W)
    cbt = codebook[:_CODES].T
    z, idx2 = _k1(e0, e1, mlp_w1, mlp_b1, mlp_w2, mlp_b2, mlp_ln_g, mlp_ln_b,
                  proj_in_w, proj_in_b, cbt)
    idx = idx2[:, 0]

    idxp = jnp.concatenate(
        [idx, jnp.full((_ROWS_PAD - _ROWS,), _CODES, jnp.int32)])
    qfp = _gather_codes(codebook, idxp)

    zp = jnp.concatenate([z, jnp.zeros((_ROWS_PAD - _ROWS, _CBD), jnp.float32)])
    quant, closs_parts = _k3(qfp, zp, proj_out_w, proj_out_b)
    commit_loss = jnp.sum(closs_parts[:, 0, 0]) / (_ROWS * _CBD)
    mv = quant[_ROWS:_ROWS + 1]

    up = jnp.repeat(quant[:_ROWS].reshape(_B, _T // 2, _W), 2, axis=1)
    upp = jnp.pad(up, ((0, 0), (0, _TP - _T), (0, 0)))
    maskf = jnp.pad(mask.astype(jnp.float32), ((0, 0), (0, _TP - _T)))[..., None]
    posp = jnp.pad(pos_emb, ((0, _TP - _T), (0, 0)))

    xa, q, k, v = _k4(upp, maskf, posp, mv, attn_ln_g, attn_ln_b,
                      wq, bq, wk, wv, bv)

    q4 = q.reshape(_B, _TP, _NH, _HW).transpose(0, 2, 1, 3)
    k4 = k.reshape(_B, _TP, _NH, _HW).transpose(0, 2, 1, 3)
    v4 = v.reshape(_B, _TP, _NH, _HW).transpose(0, 2, 1, 3)
    half = _HW // 2
    freqs = 1.0 / (10000.0 ** (jnp.arange(half, dtype=jnp.float32) / half))
    ang = jnp.arange(_TP, dtype=jnp.float32)[:, None] * freqs[None, :]
    o4 = _k5(q4, k4, v4, jnp.cos(ang), jnp.sin(ang))
    o = o4.transpose(0, 2, 1, 3).reshape(_B, _TP, _W)

    out = _k6(xa, o, wo, bo, ffn_ln_g, ffn_ln_b, ffn_w1, ffn_b1,
              ffn_w2, ffn_b2, ln_post_g, ln_post_b)

    return out[:, :_T, :], idx.reshape(_B, _T // 2), commit_loss


# speedup vs baseline: 3.0662x; 3.0662x over previous
import functools
import math

import jax
import jax.numpy as jnp
from jax import lax
from jax.experimental import pallas as pl
from jax.experimental.pallas import tpu as pltpu
from jax.experimental.pallas import tpu_sc as plsc

_B = 16
_T = 1500
_TP = 1536
_W = 512
_NH = 8
_HW = 64
_NMLP = 2048
_CODES = 512
_CBD = 32
_QK_SCALE = 2.0 * 8 / math.sqrt(_HW)
_ROWS = _B * _T // 2
_ROWS_PAD = 12032
_NW = 32
_BPW = _ROWS_PAD // _NW

_HI = lax.Precision.HIGHEST


def _gelu(x):
    return 0.5 * x * (1.0 + lax.erf(x * (1.0 / math.sqrt(2.0))))


def _ln_rows(x, g, b):
    m = jnp.mean(x, axis=-1, keepdims=True)
    xc = x - m
    v = jnp.mean(xc * xc, axis=-1, keepdims=True)
    return xc / jnp.sqrt(v + 1e-5) * g + b


def _k1_body(e0_ref, e1_ref, w1_ref, b1_ref, w2_ref, b2_ref, g_ref, bb_ref,
             pw_ref, pb_ref, cbt_ref, z_ref, idx_ref):
    x = (e0_ref[...] + e1_ref[...]) * 0.5
    h = _ln_rows(x, g_ref[...], bb_ref[...])
    a = jnp.dot(h, w1_ref[...]) + b1_ref[...]
    a = _gelu(a)
    x = x + jnp.dot(a, w2_ref[...]) + b2_ref[...]
    z = jnp.dot(x, pw_ref[...]) + pb_ref[...]
    cbt = cbt_ref[...]
    zc = jnp.dot(z, cbt)
    cb2 = jnp.sum(cbt * cbt, axis=0, keepdims=True)
    d = cb2 - 2.0 * zc
    md = jnp.min(d, axis=1, keepdims=True)
    ii = lax.broadcasted_iota(jnp.int32, d.shape, 1)
    idx = jnp.min(jnp.where(d == md, ii, _CODES), axis=1)
    z_ref[...] = z
    idx_ref[...] = idx[:, None]


def _k1(e0, e1, mlp_w1, mlp_b1, mlp_w2, mlp_b2, mlp_ln_g, mlp_ln_b,
        proj_in_w, proj_in_b, cbt):
    rt = 600
    ntile = _ROWS // rt
    full = lambda shape: pl.BlockSpec(shape, lambda i: (0,) * len(shape))
    return pl.pallas_call(
        _k1_body,
        grid=(ntile,),
        in_specs=[
            pl.BlockSpec((rt, _W), lambda i: (i, 0)),
            pl.BlockSpec((rt, _W), lambda i: (i, 0)),
            full((_W, _NMLP)),
            full((1, _NMLP)),
            full((_NMLP, _W)),
            full((1, _W)),
            full((1, _W)),
            full((1, _W)),
            full((_W, _CBD)),
            full((1, _CBD)),
            full((_CBD, _CODES)),
        ],
        out_specs=[
            pl.BlockSpec((rt, _CBD), lambda i: (i, 0)),
            pl.BlockSpec((rt, 1), lambda i: (i, 0)),
        ],
        out_shape=[
            jax.ShapeDtypeStruct((_ROWS, _CBD), jnp.float32),
            jax.ShapeDtypeStruct((_ROWS, 1), jnp.int32),
        ],
    )(e0, e1, mlp_w1, mlp_b1[None, :], mlp_w2, mlp_b2[None, :],
      mlp_ln_g[None, :], mlp_ln_b[None, :], proj_in_w, proj_in_b[None, :],
      cbt)


_GW = 128


def _sc_gather_body(table_hbm, idx_hbm, out_hbm, idx_v, rows_v, sem):
    wid = lax.axis_index("s") * 2 + lax.axis_index("c")
    base = wid * _BPW
    pltpu.sync_copy(idx_hbm.at[pl.ds(base, _BPW)], idx_v)
    pltpu.async_copy(table_hbm.at[idx_v], rows_v, sem).wait()
    pltpu.sync_copy(rows_v, out_hbm.at[pl.ds(base, _BPW)])


def _gather_codes(codebook, idxp):
    table = jnp.pad(codebook, ((0, 0), (0, _GW - _CBD)))
    mesh = plsc.VectorSubcoreMesh(core_axis_name="c", subcore_axis_name="s")
    fn = pl.kernel(
        _sc_gather_body,
        out_type=jax.ShapeDtypeStruct((_ROWS_PAD, _GW), jnp.float32),
        mesh=mesh,
        scratch_types=[
            pltpu.VMEM((_BPW,), jnp.int32),
            pltpu.VMEM((_BPW, _GW), jnp.float32),
            pltpu.SemaphoreType.DMA,
        ],
    )
    return fn(table, idxp)[:, :_CBD]


def _k3_body(qf_ref, z_ref, wpo_ref, pbo_ref, quant_ref, closs_ref):
    i = pl.program_id(0)
    qf = qf_ref[...]
    quant_ref[...] = jnp.dot(qf, wpo_ref[...]) + pbo_ref[...]
    rt = qf.shape[0]
    rows = i * rt + lax.broadcasted_iota(jnp.int32, (rt, 1), 0)
    w = (rows < _ROWS).astype(jnp.float32)
    dz = z_ref[...] - qf
    c = jnp.sum(w * dz * dz)
    closs_ref[...] = jnp.broadcast_to(c, (1, 1, 128))


def _k3(qfp, zp, proj_out_w, proj_out_b):
    rt = 752
    ntile = _ROWS_PAD // rt
    return pl.pallas_call(
        _k3_body,
        grid=(ntile,),
        in_specs=[
            pl.BlockSpec((rt, _CBD), lambda i: (i, 0)),
            pl.BlockSpec((rt, _CBD), lambda i: (i, 0)),
            pl.BlockSpec((_CBD, _W), lambda i: (0, 0)),
            pl.BlockSpec((1, _W), lambda i: (0, 0)),
        ],
        out_specs=[
            pl.BlockSpec((rt, _W), lambda i: (i, 0)),
            pl.BlockSpec((1, 1, 128), lambda i: (i, 0, 0)),
        ],
        out_shape=[
            jax.ShapeDtypeStruct((_ROWS_PAD, _W), jnp.float32),
            jax.ShapeDtypeStruct((ntile, 1, 128), jnp.float32),
        ],
    )(qfp, zp, proj_out_w, proj_out_b[None, :])


def _k4_body(up_ref, mk_ref, pos_ref, mv_ref, g_ref, b_ref,
             wq_ref, bq_ref, wk_ref, wv_ref, bv_ref,
             xa_ref, q_ref, k_ref, v_ref):
    up = up_ref[0]
    mk = mk_ref[0]
    xa = jnp.where(mk > 0, up, mv_ref[...]) + pos_ref[...]
    xa_ref[0] = xa
    h = _ln_rows(xa, g_ref[...], b_ref[...])
    q_ref[0] = (jnp.dot(h, wq_ref[...]) + bq_ref[...]) * _QK_SCALE
    k_ref[0] = jnp.dot(h, wk_ref[...])
    v_ref[0] = jnp.dot(h, wv_ref[...]) + bv_ref[...]


def _k4(upp, maskf, posp, mv, attn_ln_g, attn_ln_b, wq, bq, wk, wv, bv):
    tt = 512
    nt = _TP // tt
    full2 = lambda shape: pl.BlockSpec(shape, lambda b, j: (0,) * len(shape))
    o_spec = pl.BlockSpec((1, tt, _W), lambda b, j: (b, j, 0))
    o_shape = jax.ShapeDtypeStruct((_B, _TP, _W), jnp.float32)
    return pl.pallas_call(
        _k4_body,
        grid=(_B, nt),
        in_specs=[
            pl.BlockSpec((1, tt, _W), lambda b, j: (b, j, 0)),
            pl.BlockSpec((1, tt, 1), lambda b, j: (b, j, 0)),
            pl.BlockSpec((tt, _W), lambda b, j: (j, 0)),
            full2((1, _W)),
            full2((1, _W)),
            full2((1, _W)),
            full2((_W, _W)),
            full2((1, _W)),
            full2((_W, _W)),
            full2((_W, _W)),
            full2((1, _W)),
        ],
        out_specs=[o_spec, o_spec, o_spec, o_spec],
        out_shape=[o_shape, o_shape, o_shape, o_shape],
    )(upp, maskf, posp, mv, attn_ln_g[None, :], attn_ln_b[None, :],
      wq, bq[None, :], wk, wv, bv[None, :])


def _k5_body(q_ref, k_ref, v_ref, cos_ref, sin_ref, o_ref):
    cos = cos_ref[...]
    sin = sin_ref[...]

    def rope(x):
        x1 = x[:, :_HW // 2]
        x2 = x[:, _HW // 2:]
        return jnp.concatenate([x1 * cos - x2 * sin, x1 * sin + x2 * cos],
                               axis=1)

    qr = rope(q_ref[0, 0])
    kr = rope(k_ref[0, 0])
    s = jnp.dot(qr, kr.T)
    col = lax.broadcasted_iota(jnp.int32, s.shape, 1)
    s = jnp.where(col < _T, s, -1e30)
    m = jnp.max(s, axis=1, keepdims=True)
    p = jnp.exp(s - m)
    ps = jnp.sum(p, axis=1, keepdims=True)
    o = jnp.dot(p, v_ref[0, 0])
    o_ref[0, 0] = o / ps


def _k5(q4, k4, v4, cos, sin):
    spec = pl.BlockSpec((1, 1, _TP, _HW), lambda b, h: (b, h, 0, 0))
    return pl.pallas_call(
        _k5_body,
        grid=(_B, _NH),
        in_specs=[
            spec, spec, spec,
            pl.BlockSpec((_TP, _HW // 2), lambda b, h: (0, 0)),
            pl.BlockSpec((_TP, _HW // 2), lambda b, h: (0, 0)),
        ],
        out_specs=spec,
        out_shape=jax.ShapeDtypeStruct((_B, _NH, _TP, _HW), jnp.float32),
    )(q4, k4, v4, cos, sin)


def _k6_body(xa_ref, o_ref, wo_ref, bo_ref, g1_ref, b1_ref,
             fw1_ref, fb1_ref, fw2_ref, fb2_ref, gp_ref, bp_ref, out_ref):
    x = xa_ref[0] + jnp.dot(o_ref[0], wo_ref[...]) + bo_ref[...]
    h = _ln_rows(x, g1_ref[...], b1_ref[...])
    a = _gelu(jnp.dot(h, fw1_ref[...]) + fb1_ref[...])
    x = x + jnp.dot(a, fw2_ref[...]) + fb2_ref[...]
    out_ref[0] = _ln_rows(x, gp_ref[...], bp_ref[...])


def _k6(xa, o, wo, bo, ffn_ln_g, ffn_ln_b, ffn_w1, ffn_b1, ffn_w2, ffn_b2,
        ln_post_g, ln_post_b):
    tt = 512
    nt = _TP // tt
    full2 = lambda shape: pl.BlockSpec(shape, lambda b, j: (0,) * len(shape))
    io_spec = pl.BlockSpec((1, tt, _W), lambda b, j: (b, j, 0))
    return pl.pallas_call(
        _k6_body,
        grid=(_B, nt),
        in_specs=[
            io_spec, io_spec,
            full2((_W, _W)),
            full2((1, _W)),
            full2((1, _W)),
            full2((1, _W)),
            full2((_W, _NMLP)),
            full2((1, _NMLP)),
            full2((_NMLP, _W)),
            full2((1, _W)),
            full2((1, _W)),
            full2((1, _W)),
        ],
        out_specs=io_spec,
        out_shape=jax.ShapeDtypeStruct((_B, _TP, _W), jnp.float32),
    )(xa, o, wo, bo[None, :], ffn_ln_g[None, :], ffn_ln_b[None, :],
      ffn_w1, ffn_b1[None, :], ffn_w2, ffn_b2[None, :],
      ln_post_g[None, :], ln_post_b[None, :])


def kernel(embs, mask, mlp_w1, mlp_b1, mlp_w2, mlp_b2, mlp_ln_g, mlp_ln_b,
           proj_in_w, proj_in_b, codebook, proj_out_w, proj_out_b, pos_emb,
           attn_ln_g, attn_ln_b, wq, bq, wk, wv, bv, wo, bo,
           ffn_ln_g, ffn_ln_b, ffn_w1, ffn_b1, ffn_w2, ffn_b2,
           ln_post_g, ln_post_b):
    e0 = embs[:, 0::2, :].reshape(_ROWS, _W)
    e1 = embs[:, 1::2, :].reshape(_ROWS, _W)
    cbt = codebook[:_CODES].T
    z, idx2 = _k1(e0, e1, mlp_w1, mlp_b1, mlp_w2, mlp_b2, mlp_ln_g, mlp_ln_b,
                  proj_in_w, proj_in_b, cbt)
    idx = idx2[:, 0]

    idxp = jnp.concatenate(
        [idx, jnp.full((_ROWS_PAD - _ROWS,), _CODES, jnp.int32)])
    qfp = _gather_codes(codebook, idxp)

    zp = jnp.concatenate([z, jnp.zeros((_ROWS_PAD - _ROWS, _CBD), jnp.float32)])
    quant, closs_parts = _k3(qfp, zp, proj_out_w, proj_out_b)
    commit_loss = jnp.sum(closs_parts[:, 0, 0]) / (_ROWS * _CBD)
    mv = quant[_ROWS:_ROWS + 1]

    up = jnp.repeat(quant[:_ROWS].reshape(_B, _T // 2, _W), 2, axis=1)
    upp = jnp.pad(up, ((0, 0), (0, _TP - _T), (0, 0)))
    maskf = jnp.pad(mask.astype(jnp.float32), ((0, 0), (0, _TP - _T)))[..., None]
    posp = jnp.pad(pos_emb, ((0, _TP - _T), (0, 0)))

    xa, q, k, v = _k4(upp, maskf, posp, mv, attn_ln_g, attn_ln_b,
                      wq, bq, wk, wv, bv)

    q4 = q.reshape(_B, _TP, _NH, _HW).transpose(0, 2, 1, 3)
    k4 = k.reshape(_B, _TP, _NH, _HW).transpose(0, 2, 1, 3)
    v4 = v.reshape(_B, _TP, _NH, _HW).transpose(0, 2, 1, 3)
    half = _HW // 2
    freqs = 1.0 / (10000.0 ** (jnp.arange(half, dtype=jnp.float32) / half))
    ang = jnp.arange(_TP, dtype=jnp.float32)[:, None] * freqs[None, :]
    o4 = _k5(q4, k4, v4, jnp.cos(ang), jnp.sin(ang))
    o = o4.transpose(0, 2, 1, 3).reshape(_B, _TP, _W)

    out = _k6(xa, o, wo, bo, ffn_ln_g, ffn_ln_b, ffn_w1, ffn_b1,
              ffn_w2, ffn_b2, ln_post_g, ln_post_b)

    return out[:, :_T, :], idx.reshape(_B, _T // 2), commit_loss


# speedup vs baseline: 4.3459x; 1.4174x over previous
import functools
import math

import jax
import jax.numpy as jnp
from jax import lax
from jax.experimental import pallas as pl
from jax.experimental.pallas import tpu as pltpu
from jax.experimental.pallas import tpu_sc as plsc

_B = 16
_T = 1500
_TP = 1536
_TD = 750
_TDP = 768
_W = 512
_NH = 8
_HW = 64
_NMLP = 2048
_CODES = 512
_CBD = 32
_QK_SCALE = 2.0 * 8 / math.sqrt(_HW)
_ROWS = _B * _TD
_ROWS_PAD = _B * _TDP
_NW = 32
_BPW = _ROWS_PAD // _NW


def _gelu(x):
    return 0.5 * x * (1.0 + lax.erf(x * (1.0 / math.sqrt(2.0))))


def _ln_rows(x, g, b):
    m = jnp.mean(x, axis=-1, keepdims=True)
    xc = x - m
    v = jnp.mean(xc * xc, axis=-1, keepdims=True)
    return xc / jnp.sqrt(v + 1e-5) * g + b


def _k1_body(e0_ref, e1_ref, w1_ref, b1_ref, w2_ref, b2_ref, g_ref, bb_ref,
             pw_ref, pb_ref, cbt_ref, z_ref, idx_ref):
    x = (e0_ref[...] + e1_ref[...]) * 0.5
    h = _ln_rows(x, g_ref[...], bb_ref[...])
    a = _gelu(jnp.dot(h, w1_ref[...]) + b1_ref[...])
    x = x + jnp.dot(a, w2_ref[...]) + b2_ref[...]
    z = jnp.dot(x, pw_ref[...]) + pb_ref[...]
    cbt = cbt_ref[...]
    zc = jnp.dot(z, cbt)
    cb2 = jnp.sum(cbt * cbt, axis=0, keepdims=True)
    d = cb2 - 2.0 * zc
    md = jnp.min(d, axis=1, keepdims=True)
    ii = lax.broadcasted_iota(jnp.int32, d.shape, 1)
    idx = jnp.min(jnp.where(d == md, ii, _CODES), axis=1)
    z_ref[...] = z
    idx_ref[...] = idx[:, None]


def _k1(e0, e1, mlp_w1, mlp_b1, mlp_w2, mlp_b2, mlp_ln_g, mlp_ln_b,
        proj_in_w, proj_in_b, cbt):
    rt = 600
    ntile = _ROWS // rt
    full = lambda shape: pl.BlockSpec(shape, lambda i: (0,) * len(shape))
    return pl.pallas_call(
        _k1_body,
        grid=(ntile,),
        in_specs=[
            pl.BlockSpec((rt, _W), lambda i: (i, 0)),
            pl.BlockSpec((rt, _W), lambda i: (i, 0)),
            full((_W, _NMLP)),
            full((1, _NMLP)),
            full((_NMLP, _W)),
            full((1, _W)),
            full((1, _W)),
            full((1, _W)),
            full((_W, _CBD)),
            full((1, _CBD)),
            full((_CBD, _CODES)),
        ],
        out_specs=[
            pl.BlockSpec((rt, _CBD), lambda i: (i, 0)),
            pl.BlockSpec((rt, 1), lambda i: (i, 0)),
        ],
        out_shape=[
            jax.ShapeDtypeStruct((_ROWS, _CBD), jnp.float32),
            jax.ShapeDtypeStruct((_ROWS, 1), jnp.int32),
        ],
    )(e0, e1, mlp_w1, mlp_b1[None, :], mlp_w2, mlp_b2[None, :],
      mlp_ln_g[None, :], mlp_ln_b[None, :], proj_in_w, proj_in_b[None, :],
      cbt)


_GW = 128


def _sc_gather_body(table_hbm, idx_hbm, out_hbm, idx_v, rows_v, sem):
    wid = lax.axis_index("s") * 2 + lax.axis_index("c")
    base = wid * _BPW
    pltpu.sync_copy(idx_hbm.at[pl.ds(base, _BPW)], idx_v)
    pltpu.async_copy(table_hbm.at[idx_v], rows_v, sem).wait()
    pltpu.sync_copy(rows_v, out_hbm.at[pl.ds(base, _BPW)])


def _gather_codes(codebook, idxp):
    table = jnp.pad(codebook, ((0, 0), (0, _GW - _CBD)))
    mesh = plsc.VectorSubcoreMesh(core_axis_name="c", subcore_axis_name="s")
    fn = pl.kernel(
        _sc_gather_body,
        out_type=jax.ShapeDtypeStruct((_ROWS_PAD, _GW), jnp.float32),
        mesh=mesh,
        scratch_types=[
            pltpu.VMEM((_BPW,), jnp.int32),
            pltpu.VMEM((_BPW, _GW), jnp.float32),
            pltpu.SemaphoreType.DMA,
        ],
    )
    return fn(table, idxp)[:, :_CBD]


def _k2_body(qf_ref, z_ref, closs_ref):
    rows = lax.broadcasted_iota(jnp.int32, (_TDP, 1), 0)
    w = (rows < _TD).astype(jnp.float32)
    dz = z_ref[0] - qf_ref[0]
    closs_ref[...] = jnp.broadcast_to(jnp.sum(w * dz * dz), (1, 1, 128))


def _k2(qf3, z3):
    return pl.pallas_call(
        _k2_body,
        grid=(_B,),
        in_specs=[
            pl.BlockSpec((1, _TDP, _CBD), lambda b: (b, 0, 0)),
            pl.BlockSpec((1, _TDP, _CBD), lambda b: (b, 0, 0)),
        ],
        out_specs=pl.BlockSpec((1, 1, 128), lambda b: (b, 0, 0)),
        out_shape=jax.ShapeDtypeStruct((_B, 1, 128), jnp.float32),
    )(qf3, z3)


def _k4_body(qf_ref, mk_ref, pos_ref, cbrow_ref, wpo_ref, pbo_ref,
             g_ref, b_ref, wq_ref, bq_ref, wk_ref, wv_ref, bv_ref,
             xa_ref, q_ref, k_ref, v_ref):
    j = pl.program_id(1)
    wpo = wpo_ref[...]
    pbo = pbo_ref[...]
    qp = jnp.dot(qf_ref[0], wpo) + pbo
    up = jnp.repeat(qp, 2, axis=0)
    mv = jnp.dot(cbrow_ref[...], wpo) + pbo
    xa = jnp.where(mk_ref[0] > 0, up, mv) + pos_ref[...]
    rows = j * xa.shape[0] + lax.broadcasted_iota(jnp.int32, (xa.shape[0], 1), 0)
    xa = jnp.where(rows < _T, xa, 0.0)
    xa_ref[0] = xa
    h = _ln_rows(xa, g_ref[...], b_ref[...])
    q_ref[0] = (jnp.dot(h, wq_ref[...]) + bq_ref[...]) * _QK_SCALE
    k_ref[0] = jnp.dot(h, wk_ref[...])
    v_ref[0] = jnp.dot(h, wv_ref[...]) + bv_ref[...]


def _k4(qf3, maskf, posp, cbrow, proj_out_w, proj_out_b,
        attn_ln_g, attn_ln_b, wq, bq, wk, wv, bv):
    tt = 512
    nt = _TP // tt
    full2 = lambda shape: pl.BlockSpec(shape, lambda b, j: (0,) * len(shape))
    o_spec = pl.BlockSpec((1, tt, _W), lambda b, j: (b, j, 0))
    o_shape = jax.ShapeDtypeStruct((_B, _TP, _W), jnp.float32)
    return pl.pallas_call(
        _k4_body,
        grid=(_B, nt),
        in_specs=[
            pl.BlockSpec((1, tt // 2, _CBD), lambda b, j: (b, j, 0)),
            pl.BlockSpec((1, tt, 1), lambda b, j: (b, j, 0)),
            pl.BlockSpec((tt, _W), lambda b, j: (j, 0)),
            full2((1, _CBD)),
            full2((_CBD, _W)),
            full2((1, _W)),
            full2((1, _W)),
            full2((1, _W)),
            full2((_W, _W)),
            full2((1, _W)),
            full2((_W, _W)),
            full2((_W, _W)),
            full2((1, _W)),
        ],
        out_specs=[o_spec, o_spec, o_spec, o_spec],
        out_shape=[o_shape, o_shape, o_shape, o_shape],
    )(qf3, maskf, posp, cbrow, proj_out_w, proj_out_b[None, :],
      attn_ln_g[None, :], attn_ln_b[None, :], wq, bq[None, :], wk, wv,
      bv[None, :])


def _k5_body(q_ref, k_ref, v_ref, cos_ref, sin_ref, o_ref):
    cos = cos_ref[...]
    sin = sin_ref[...]

    def rope(x):
        x1 = x[:, :_HW // 2]
        x2 = x[:, _HW // 2:]
        return jnp.concatenate([x1 * cos - x2 * sin, x1 * sin + x2 * cos],
                               axis=1)

    outs = []
    for hh in range(2):
        sl = slice(hh * _HW, (hh + 1) * _HW)
        qr = rope(q_ref[0][:, sl])
        kr = rope(k_ref[0][:, sl])
        s = jnp.dot(qr, kr.T)
        col = lax.broadcasted_iota(jnp.int32, s.shape, 1)
        s = jnp.where(col < _T, s, -1e30)
        m = jnp.max(s, axis=1, keepdims=True)
        p = jnp.exp(s - m)
        ps = jnp.sum(p, axis=1, keepdims=True)
        outs.append(jnp.dot(p, v_ref[0][:, sl]) / ps)
    o_ref[0] = jnp.concatenate(outs, axis=1)


def _k5(q, k, v, cos, sin):
    spec = pl.BlockSpec((1, _TP, 2 * _HW), lambda b, g: (b, 0, g))
    return pl.pallas_call(
        _k5_body,
        grid=(_B, _NH // 2),
        in_specs=[
            spec, spec, spec,
            pl.BlockSpec((_TP, _HW // 2), lambda b, g: (0, 0)),
            pl.BlockSpec((_TP, _HW // 2), lambda b, g: (0, 0)),
        ],
        out_specs=spec,
        out_shape=jax.ShapeDtypeStruct((_B, _TP, _W), jnp.float32),
    )(q, k, v, cos, sin)


def _k6_body(xa_ref, o_ref, wo_ref, bo_ref, g1_ref, b1_ref,
             fw1_ref, fb1_ref, fw2_ref, fb2_ref, gp_ref, bp_ref, out_ref):
    x = xa_ref[0] + jnp.dot(o_ref[0], wo_ref[...]) + bo_ref[...]
    h = _ln_rows(x, g1_ref[...], b1_ref[...])
    a = _gelu(jnp.dot(h, fw1_ref[...]) + fb1_ref[...])
    x = x + jnp.dot(a, fw2_ref[...]) + fb2_ref[...]
    out_ref[0] = _ln_rows(x, gp_ref[...], bp_ref[...])


def _k6(xa, o, wo, bo, ffn_ln_g, ffn_ln_b, ffn_w1, ffn_b1, ffn_w2, ffn_b2,
        ln_post_g, ln_post_b):
    tt = 512
    nt = _TP // tt
    full2 = lambda shape: pl.BlockSpec(shape, lambda b, j: (0,) * len(shape))
    io_spec = pl.BlockSpec((1, tt, _W), lambda b, j: (b, j, 0))
    return pl.pallas_call(
        _k6_body,
        grid=(_B, nt),
        in_specs=[
            io_spec, io_spec,
            full2((_W, _W)),
            full2((1, _W)),
            full2((1, _W)),
            full2((1, _W)),
            full2((_W, _NMLP)),
            full2((1, _NMLP)),
            full2((_NMLP, _W)),
            full2((1, _W)),
            full2((1, _W)),
            full2((1, _W)),
        ],
        out_specs=io_spec,
        out_shape=jax.ShapeDtypeStruct((_B, _TP, _W), jnp.float32),
    )(xa, o, wo, bo[None, :], ffn_ln_g[None, :], ffn_ln_b[None, :],
      ffn_w1, ffn_b1[None, :], ffn_w2, ffn_b2[None, :],
      ln_post_g[None, :], ln_post_b[None, :])


def kernel(embs, mask, mlp_w1, mlp_b1, mlp_w2, mlp_b2, mlp_ln_g, mlp_ln_b,
           proj_in_w, proj_in_b, codebook, proj_out_w, proj_out_b, pos_emb,
           attn_ln_g, attn_ln_b, wq, bq, wk, wv, bv, wo, bo,
           ffn_ln_g, ffn_ln_b, ffn_w1, ffn_b1, ffn_w2, ffn_b2,
           ln_post_g, ln_post_b):
    e0 = embs[:, 0::2, :].reshape(_ROWS, _W)
    e1 = embs[:, 1::2, :].reshape(_ROWS, _W)
    cbt = codebook[:_CODES].T
    z, idx2 = _k1(e0, e1, mlp_w1, mlp_b1, mlp_w2, mlp_b2, mlp_ln_g, mlp_ln_b,
                  proj_in_w, proj_in_b, cbt)
    idx = idx2[:, 0]

    idxp = jnp.pad(idx.reshape(_B, _TD), ((0, 0), (0, _TDP - _TD)),
                   constant_values=_CODES).reshape(_ROWS_PAD)
    qfp = _gather_codes(codebook, idxp)
    qf3 = qfp.reshape(_B, _TDP, _CBD)

    z3 = jnp.pad(z.reshape(_B, _TD, _CBD), ((0, 0), (0, _TDP - _TD), (0, 0)))
    closs_parts = _k2(qf3, z3)
    commit_loss = jnp.sum(closs_parts[:, 0, 0]) / (_ROWS * _CBD)

    maskf = jnp.pad(mask.astype(jnp.float32), ((0, 0), (0, _TP - _T)))[..., None]
    posp = jnp.pad(pos_emb, ((0, _TP - _T), (0, 0)))
    xa, q, k, v = _k4(qf3, maskf, posp, codebook[_CODES:_CODES + 1],
                      proj_out_w, proj_out_b, attn_ln_g, attn_ln_b,
                      wq, bq, wk, wv, bv)

    half = _HW // 2
    freqs = 1.0 / (10000.0 ** (jnp.arange(half, dtype=jnp.float32) / half))
    ang = jnp.arange(_TP, dtype=jnp.float32)[:, None] * freqs[None, :]
    o = _k5(q, k, v, jnp.cos(ang), jnp.sin(ang))

    out = _k6(xa, o, wo, bo, ffn_ln_g, ffn_ln_b, ffn_w1, ffn_b1,
              ffn_w2, ffn_b2, ln_post_g, ln_post_b)

    return out[:, :_T, :], idx.reshape(_B, _TD), commit_loss
